# Initial kernel scaffold; baseline (speedup 1.0000x reference)
import jax
import jax.numpy as jnp
from jax.experimental import pallas as pl


def kernel(x, edge_index, W1, b1, gamma, beta, W2, b2):
    raise NotImplementedError("write your pallas kernel here")


# speedup vs baseline: 13.1557x; 13.1557x over previous
import functools

import jax
import jax.numpy as jnp
from jax import lax
from jax.experimental import pallas as pl
from jax.experimental.pallas import tpu as pltpu
from jax.experimental.pallas import tpu_sc as plsc

N = 10000
E = 320000
D = 128

NC = 2
NS = 16
NW = NC * NS

CHUNK = 128
K = -(-E // (NW * CHUNK))
E_PAD = NW * K * CHUNK
EPT = K * CHUNK

ACC_ROWS = 10112
ZPT = ACC_ROWS // NS

BLK = 1000
INV_BN = 0.9999950000374997

_MESH = plsc.VectorSubcoreMesh(core_axis_name="c", subcore_axis_name="s")


@functools.partial(
    pl.kernel,
    out_type=jax.ShapeDtypeStruct((NC * ACC_ROWS,), jnp.float32),
    mesh=_MESH,
    scratch_types=[
        pltpu.VMEM((K, CHUNK), jnp.int32),
        pltpu.VMEM((CHUNK,), jnp.float32),
        pltpu.VMEM((ZPT,), jnp.float32),
        pltpu.VMEM_SHARED((ACC_ROWS,), jnp.float32),
    ],
)
def _deg_kernel(dst_hbm, ones_hbm, zeros_hbm, deg_hbm, idx_d, ones_v, stage, acc):
    c = lax.axis_index("c")
    s = lax.axis_index("s")
    wid = c * NS + s
    pltpu.sync_copy(zeros_hbm.at[pl.ds(0, ZPT)], stage)
    pltpu.sync_copy(stage, acc.at[pl.ds(s * ZPT, ZPT)])
    pltpu.sync_copy(ones_hbm, ones_v)
    pltpu.sync_copy(dst_hbm.at[wid], idx_d)
    plsc.subcore_barrier()

    def body(j, carry):
        pltpu.sync_copy(ones_v, acc.at[idx_d.at[j]], add=True)
        return carry

    lax.fori_loop(0, K, body, 0)
    plsc.subcore_barrier()
    pltpu.sync_copy(acc.at[pl.ds(s * ZPT, ZPT)], stage)
    pltpu.sync_copy(stage, deg_hbm.at[pl.ds(c * ACC_ROWS + s * ZPT, ZPT)])


@functools.partial(
    pl.kernel,
    out_type=jax.ShapeDtypeStruct((NC, ACC_ROWS, D), jnp.float32),
    mesh=_MESH,
    scratch_types=[
        pltpu.VMEM((K, CHUNK), jnp.int32),
        pltpu.VMEM((K, CHUNK), jnp.int32),
        pltpu.VMEM((CHUNK, D), jnp.float32),
        pltpu.VMEM_SHARED((ACC_ROWS, D), jnp.float32),
    ],
)
def _agg_kernel(hs_hbm, src_hbm, dst_hbm, zeros_hbm, out_hbm,
                idx_s, idx_d, rows, acc):
    c = lax.axis_index("c")
    s = lax.axis_index("s")
    wid = c * NS + s
    pieces = [(o, min(CHUNK, ZPT - o)) for o in range(0, ZPT, CHUNK)]
    pltpu.sync_copy(zeros_hbm, rows)
    for off, sz in pieces:
        pltpu.sync_copy(rows.at[pl.ds(0, sz)], acc.at[pl.ds(s * ZPT + off, sz)])
    pltpu.sync_copy(src_hbm.at[wid], idx_s)
    pltpu.sync_copy(dst_hbm.at[wid], idx_d)
    plsc.subcore_barrier()

    def body(j, carry):
        pltpu.sync_copy(hs_hbm.at[idx_s.at[j]], rows)
        pltpu.sync_copy(rows, acc.at[idx_d.at[j]], add=True)
        return carry

    lax.fori_loop(0, K, body, 0)
    plsc.subcore_barrier()
    for off, sz in pieces:
        pltpu.sync_copy(acc.at[pl.ds(s * ZPT + off, sz)], rows.at[pl.ds(0, sz)])
        pltpu.sync_copy(rows.at[pl.ds(0, sz)],
                        out_hbm.at[c, pl.ds(s * ZPT + off, sz)])


def _dense1_body(deg_ref, x_ref, w_ref, o_ref):
    dis = lax.rsqrt(deg_ref[0] + deg_ref[1] + 1.0)
    h = jnp.dot(x_ref[...], w_ref[...], preferred_element_type=jnp.float32)
    o_ref[...] = h * dis


def _dense2_body(deg_ref, a_ref, hs1_ref, b1_ref, g_ref, be_ref, w2_ref, o_ref):
    dis = lax.rsqrt(deg_ref[0] + deg_ref[1] + 1.0)
    u = (a_ref[0] + a_ref[1] + hs1_ref[...]) * dis + b1_ref[...]
    u = u * (g_ref[...] * INV_BN) + be_ref[...]
    u = jnp.maximum(u, 0.0)
    h = jnp.dot(u, w2_ref[...], preferred_element_type=jnp.float32)
    o_ref[...] = h * dis


def _dense3_body(deg_ref, a_ref, hs2_ref, b2_ref, o_ref):
    dis = lax.rsqrt(deg_ref[0] + deg_ref[1] + 1.0)
    o_ref[...] = (a_ref[0] + a_ref[1] + hs2_ref[...]) * dis + b2_ref[...]


_DEG_SPEC = pl.BlockSpec((NC, BLK, 1), lambda i: (0, i, 0))
_ROW_SPEC = pl.BlockSpec((BLK, D), lambda i: (i, 0))
_AGG_SPEC = pl.BlockSpec((NC, BLK, D), lambda i: (0, i, 0))
_W_SPEC = pl.BlockSpec((D, D), lambda i: (0, 0))
_VEC_SPEC = pl.BlockSpec((1, D), lambda i: (0, 0))


def _dense1(deg3, x, W1):
    return pl.pallas_call(
        _dense1_body,
        grid=(N // BLK,),
        in_specs=[_DEG_SPEC, _ROW_SPEC, _W_SPEC],
        out_specs=_ROW_SPEC,
        out_shape=jax.ShapeDtypeStruct((N, D), jnp.float32),
    )(deg3, x, W1)


def _dense2(deg3, agg1, hs1, b1, g, be, W2):
    return pl.pallas_call(
        _dense2_body,
        grid=(N // BLK,),
        in_specs=[_DEG_SPEC, _AGG_SPEC, _ROW_SPEC, _VEC_SPEC, _VEC_SPEC,
                  _VEC_SPEC, _W_SPEC],
        out_specs=_ROW_SPEC,
        out_shape=jax.ShapeDtypeStruct((N, D), jnp.float32),
    )(deg3, agg1, hs1, b1, g, be, W2)


def _dense3(deg3, agg2, hs2, b2):
    return pl.pallas_call(
        _dense3_body,
        grid=(N // BLK,),
        in_specs=[_DEG_SPEC, _AGG_SPEC, _ROW_SPEC, _VEC_SPEC],
        out_specs=_ROW_SPEC,
        out_shape=jax.ShapeDtypeStruct((N, D), jnp.float32),
    )(deg3, agg2, hs2, b2)


def kernel(x, edge_index, W1, b1, gamma, beta, W2, b2):
    src = edge_index[0]
    dst = edge_index[1]
    pad = E_PAD - E
    src_t = jnp.concatenate(
        [src, jnp.zeros((pad,), jnp.int32)]).reshape(NW, K, CHUNK)
    dst_t = jnp.concatenate(
        [dst, jnp.full((pad,), N, jnp.int32)]).reshape(NW, K, CHUNK)
    ones_h = jnp.ones((CHUNK,), jnp.float32)
    zeros1 = jnp.zeros((ZPT,), jnp.float32)
    zeros2 = jnp.zeros((CHUNK, D), jnp.float32)

    deg_p = _deg_kernel(dst_t, ones_h, zeros1)
    deg3 = deg_p.reshape(NC, ACC_ROWS, 1)

    hs1 = _dense1(deg3, x, W1)
    agg1 = _agg_kernel(hs1, src_t, dst_t, zeros2)
    hs2 = _dense2(deg3, agg1, hs1, b1.reshape(1, D),
                  gamma.reshape(1, D), beta.reshape(1, D), W2)
    agg2 = _agg_kernel(hs2, src_t, dst_t, zeros2)
    out = _dense3(deg3, agg2, hs2, b2.reshape(1, D))
    return out


# speedup vs baseline: 14.0065x; 1.0647x over previous
import functools

import jax
import jax.numpy as jnp
from jax import lax
from jax.experimental import pallas as pl
from jax.experimental.pallas import tpu as pltpu
from jax.experimental.pallas import tpu_sc as plsc

N = 10000
E = 320000
D = 128
HD = D // 2

NC = 2
NS = 16
NW = NC * NS

CHUNK = 128

KD = -(-E // (NW * CHUNK))
E_PAD_D = NW * KD * CHUNK

KA = 159
E_PAD_A = NS * KA * CHUNK

ACC_ROWS = 10112
ZPT = ACC_ROWS // NS

BLK = 1000
INV_BN = 0.9999950000374997

_MESH = plsc.VectorSubcoreMesh(core_axis_name="c", subcore_axis_name="s")


@functools.partial(
    pl.kernel,
    out_type=jax.ShapeDtypeStruct((NC * ACC_ROWS,), jnp.float32),
    mesh=_MESH,
    scratch_types=[
        pltpu.VMEM((KD, CHUNK), jnp.int32),
        pltpu.VMEM((CHUNK,), jnp.float32),
        pltpu.VMEM((ZPT,), jnp.float32),
        pltpu.VMEM_SHARED((ACC_ROWS,), jnp.float32),
    ],
)
def _deg_kernel(dst_hbm, ones_hbm, zeros_hbm, deg_hbm, idx_d, ones_v, stage, acc):
    c = lax.axis_index("c")
    s = lax.axis_index("s")
    wid = c * NS + s
    pltpu.sync_copy(zeros_hbm.at[pl.ds(0, ZPT)], stage)
    pltpu.sync_copy(stage, acc.at[pl.ds(s * ZPT, ZPT)])
    pltpu.sync_copy(ones_hbm, ones_v)
    pltpu.sync_copy(dst_hbm.at[wid], idx_d)
    plsc.subcore_barrier()

    def body(j, carry):
        pltpu.sync_copy(ones_v, acc.at[idx_d.at[j]], add=True)
        return carry

    lax.fori_loop(0, KD, body, 0)
    plsc.subcore_barrier()
    pltpu.sync_copy(acc.at[pl.ds(s * ZPT, ZPT)], stage)
    pltpu.sync_copy(stage, deg_hbm.at[pl.ds(c * ACC_ROWS + s * ZPT, ZPT)])


@functools.partial(
    pl.kernel,
    out_type=jax.ShapeDtypeStruct((NC, ACC_ROWS, HD), jnp.float32),
    mesh=_MESH,
    scratch_types=[
        pltpu.VMEM((KA, CHUNK), jnp.int32),
        pltpu.VMEM((KA, CHUNK), jnp.int32),
        pltpu.VMEM((CHUNK, HD), jnp.float32),
        pltpu.VMEM((CHUNK, HD), jnp.float32),
        pltpu.SemaphoreType.DMA,
        pltpu.VMEM_SHARED((ACC_ROWS, HD), jnp.float32),
    ],
    compiler_params=pltpu.CompilerParams(use_tc_tiling_on_sc=False),
)
def _agg_kernel(hs_hbm, src_hbm, dst_hbm, zeros_hbm, out_hbm,
                idx_s, idx_d, rows_a, rows_b, sem, acc):
    c = lax.axis_index("c")
    s = lax.axis_index("s")
    pieces = [(o, min(CHUNK, ZPT - o)) for o in range(0, ZPT, CHUNK)]
    pltpu.sync_copy(zeros_hbm, rows_a)
    for off, sz in pieces:
        pltpu.sync_copy(rows_a.at[pl.ds(0, sz)], acc.at[pl.ds(s * ZPT + off, sz)])
    pltpu.sync_copy(src_hbm.at[c, s], idx_s)
    pltpu.sync_copy(dst_hbm.at[s], idx_d)
    plsc.subcore_barrier()

    pltpu.async_copy(hs_hbm.at[idx_s.at[0]], rows_a, sem)

    def body(i, carry):
        j = 2 * i
        pltpu.make_async_copy(hs_hbm.at[idx_s.at[j]], rows_a, sem).wait()
        pltpu.async_copy(hs_hbm.at[idx_s.at[j + 1]], rows_b, sem)
        pltpu.sync_copy(rows_a, acc.at[idx_d.at[j]], add=True)
        pltpu.make_async_copy(hs_hbm.at[idx_s.at[j + 1]], rows_b, sem).wait()
        pltpu.async_copy(hs_hbm.at[idx_s.at[j + 2]], rows_a, sem)
        pltpu.sync_copy(rows_b, acc.at[idx_d.at[j + 1]], add=True)
        return carry

    lax.fori_loop(0, (KA - 1) // 2, body, 0)
    pltpu.make_async_copy(hs_hbm.at[idx_s.at[KA - 1]], rows_a, sem).wait()
    pltpu.sync_copy(rows_a, acc.at[idx_d.at[KA - 1]], add=True)
    plsc.subcore_barrier()
    for off, sz in pieces:
        pltpu.sync_copy(acc.at[pl.ds(s * ZPT + off, sz)], rows_a.at[pl.ds(0, sz)])
        pltpu.sync_copy(rows_a.at[pl.ds(0, sz)],
                        out_hbm.at[c, pl.ds(s * ZPT + off, sz)])


def _dense1_body(deg_ref, x_ref, w_ref, o_ref):
    dis = lax.rsqrt(deg_ref[0] + deg_ref[1] + 1.0)
    h = jnp.dot(x_ref[...], w_ref[...], preferred_element_type=jnp.float32) * dis
    o_ref[0] = h[:, :HD]
    o_ref[1] = h[:, HD:]


def _dense2_body(deg_ref, a_ref, hs1_ref, b1_ref, g_ref, be_ref, w2_ref, o_ref):
    dis = lax.rsqrt(deg_ref[0] + deg_ref[1] + 1.0)
    agg = jnp.concatenate([a_ref[0] + hs1_ref[0], a_ref[1] + hs1_ref[1]], axis=1)
    u = agg * dis + b1_ref[...]
    u = u * (g_ref[...] * INV_BN) + be_ref[...]
    u = jnp.maximum(u, 0.0)
    h = jnp.dot(u, w2_ref[...], preferred_element_type=jnp.float32) * dis
    o_ref[0] = h[:, :HD]
    o_ref[1] = h[:, HD:]


def _dense3_body(deg_ref, a_ref, hs2_ref, b2_ref, o_ref):
    dis = lax.rsqrt(deg_ref[0] + deg_ref[1] + 1.0)
    agg = jnp.concatenate([a_ref[0] + hs2_ref[0], a_ref[1] + hs2_ref[1]], axis=1)
    o_ref[...] = agg * dis + b2_ref[...]


_DEG_SPEC = pl.BlockSpec((NC, BLK, 1), lambda i: (0, i, 0))
_ROW_SPEC = pl.BlockSpec((BLK, D), lambda i: (i, 0))
_HALF_SPEC = pl.BlockSpec((NC, BLK, HD), lambda i: (0, i, 0))
_W_SPEC = pl.BlockSpec((D, D), lambda i: (0, 0))
_VEC_SPEC = pl.BlockSpec((1, D), lambda i: (0, 0))


def _dense1(deg3, x, W1):
    return pl.pallas_call(
        _dense1_body,
        grid=(N // BLK,),
        in_specs=[_DEG_SPEC, _ROW_SPEC, _W_SPEC],
        out_specs=_HALF_SPEC,
        out_shape=jax.ShapeDtypeStruct((NC, N, HD), jnp.float32),
    )(deg3, x, W1)


def _dense2(deg3, agg1, hs1c, b1, g, be, W2):
    return pl.pallas_call(
        _dense2_body,
        grid=(N // BLK,),
        in_specs=[_DEG_SPEC, _HALF_SPEC, _HALF_SPEC, _VEC_SPEC, _VEC_SPEC,
                  _VEC_SPEC, _W_SPEC],
        out_specs=_HALF_SPEC,
        out_shape=jax.ShapeDtypeStruct((NC, N, HD), jnp.float32),
    )(deg3, agg1, hs1c, b1, g, be, W2)


def _dense3(deg3, agg2, hs2c, b2):
    return pl.pallas_call(
        _dense3_body,
        grid=(N // BLK,),
        in_specs=[_DEG_SPEC, _HALF_SPEC, _HALF_SPEC, _VEC_SPEC],
        out_specs=_ROW_SPEC,
        out_shape=jax.ShapeDtypeStruct((N, D), jnp.float32),
    )(deg3, agg2, hs2c, b2)


def kernel(x, edge_index, W1, b1, gamma, beta, W2, b2):
    src = edge_index[0]
    dst = edge_index[1]
    dst_deg = jnp.concatenate(
        [dst, jnp.full((E_PAD_D - E,), N, jnp.int32)]).reshape(NW, KD, CHUNK)
    src_agg = jnp.concatenate(
        [src, jnp.zeros((E_PAD_A - E,), jnp.int32)]).reshape(NS, KA, CHUNK)
    src_agg = src_agg[None] + (jnp.arange(NC, dtype=jnp.int32) * N)[:, None, None, None]
    dst_agg = jnp.concatenate(
        [dst, jnp.full((E_PAD_A - E,), N, jnp.int32)]).reshape(NS, KA, CHUNK)
    ones_h = jnp.ones((CHUNK,), jnp.float32)
    zeros1 = jnp.zeros((ZPT,), jnp.float32)
    zeros2 = jnp.zeros((CHUNK, HD), jnp.float32)

    deg_p = _deg_kernel(dst_deg, ones_h, zeros1)
    deg3 = deg_p.reshape(NC, ACC_ROWS, 1)

    hs1 = _dense1(deg3, x, W1)
    agg1 = _agg_kernel(hs1.reshape(NC * N, HD), src_agg, dst_agg, zeros2)
    hs2 = _dense2(deg3, agg1, hs1, b1.reshape(1, D),
                  gamma.reshape(1, D), beta.reshape(1, D), W2)
    agg2 = _agg_kernel(hs2.reshape(NC * N, HD), src_agg, dst_agg, zeros2)
    out = _dense3(deg3, agg2, hs2, b2.reshape(1, D))
    return out
